# baseline (speedup 1.0000x reference)
import functools

import jax
import jax.numpy as jnp
from jax import lax
from jax.experimental import pallas as pl
from jax.experimental.pallas import tpu as pltpu
from jax.experimental.pallas import tpu_sc as plsc

N = 10000
E = 320000
D = 128
H = 128

NC = 2
NS = 16
NW = NC * NS
EPW = E // NW

C1 = 128
NCH1 = 79
NCH1P = NCH1 + 1
NPAD = NCH1 * C1 - EPW
NX = N + 8
C3 = 80
NCH3 = EPW // C3
G3 = C3 // 16

ROWS_A = 632
ROWS_LAST = N - (NS - 1) * ROWS_A

_f32 = jnp.float32
_i32 = jnp.int32


@functools.partial(
    pl.kernel,
    out_type=(
        jax.ShapeDtypeStruct((NC, N, D), _f32),
        jax.ShapeDtypeStruct((NC, NX), _f32),
    ),
    mesh=plsc.VectorSubcoreMesh(core_axis_name="c", subcore_axis_name="s"),
    scratch_types=[
        pltpu.VMEM((2, C1), _i32),
        pltpu.VMEM((2, C1), _i32),
        pltpu.VMEM((C1, D), _f32),
        pltpu.VMEM((C1, D), _f32),
        pltpu.VMEM((C1,), _f32),
        pltpu.VMEM_SHARED((NX, D), _f32),
        pltpu.VMEM_SHARED((NX,), _f32),
        pltpu.SemaphoreType.DMA,
        pltpu.SemaphoreType.DMA,
        pltpu.SemaphoreType.DMA,
        pltpu.SemaphoreType.DMA,
        pltpu.SemaphoreType.DMA,
        pltpu.SemaphoreType.DMA,
        pltpu.SemaphoreType.DMA,
        pltpu.SemaphoreType.DMA,
    ],
    compiler_params=pltpu.CompilerParams(needs_layout_passes=False),
)
def _aggregate(x_hbm, rec_hbm, zeros_hbm, zeros1_hbm, ones_hbm,
               nsum_out, deg_out,
               idx_a, idx_b, rows_a, rows_b, ones_v, table_sh, deg_sh,
               sem_ia, sem_ib, sem_ga, sem_gb,
               sem_ta, sem_tb, sem_da, sem_db):
    c = lax.axis_index("c")
    s = lax.axis_index("s")
    wid = s * NC + c

    pltpu.sync_copy(ones_hbm, ones_v)

    @pl.when(s == 0)
    def _init_table():
        pltpu.sync_copy(zeros_hbm, table_sh)

    @pl.when(s == 1)
    def _init_deg():
        pltpu.sync_copy(zeros1_hbm, deg_sh)

    plsc.subcore_barrier()

    def _fetch_idx(k, idx_v, sem):
        pltpu.async_copy(rec_hbm.at[wid].at[k], idx_v, sem)

    def _wait_idx(idx_v, sem):
        pltpu.make_async_copy(rec_hbm.at[0].at[0], idx_v, sem).wait()

    def _gather(idx_v, rows_v, sem):
        pltpu.async_copy(x_hbm.at[idx_v.at[0]], rows_v, sem)

    def _wait_rows(idx_v, rows_v, sem):
        pltpu.make_async_copy(x_hbm.at[idx_v.at[0]], rows_v, sem).wait()

    def _add(idx_v, rows_v, sem_t, sem_d):
        pltpu.async_copy(rows_v, table_sh.at[idx_v.at[1]], sem_t, add=True)
        pltpu.async_copy(ones_v, deg_sh.at[idx_v.at[1]], sem_d, add=True)

    def _wait_add(idx_v, rows_v, sem_t, sem_d):
        pltpu.make_async_copy(rows_v, table_sh.at[idx_v.at[1]], sem_t).wait()
        pltpu.make_async_copy(ones_v, deg_sh.at[idx_v.at[1]], sem_d).wait()

    _fetch_idx(0, idx_a, sem_ia)
    _fetch_idx(1, idx_b, sem_ib)
    _wait_idx(idx_a, sem_ia)
    _gather(idx_a, rows_a, sem_ga)

    def _body(i, carry):
        ka = 2 * i
        _wait_idx(idx_b, sem_ib)
        _gather(idx_b, rows_b, sem_gb)
        _wait_rows(idx_a, rows_a, sem_ga)
        _add(idx_a, rows_a, sem_ta, sem_da)
        _wait_add(idx_a, rows_a, sem_ta, sem_da)
        _fetch_idx(ka + 2, idx_a, sem_ia)
        _wait_rows(idx_b, rows_b, sem_gb)
        _add(idx_b, rows_b, sem_tb, sem_db)
        _wait_add(idx_b, rows_b, sem_tb, sem_db)
        _fetch_idx(ka + 3, idx_b, sem_ib)
        _wait_idx(idx_a, sem_ia)
        _gather(idx_a, rows_a, sem_ga)
        return carry
    lax.fori_loop(0, (NCH1 - 1) // 2, _body, 0)

    _wait_rows(idx_a, rows_a, sem_ga)
    _add(idx_a, rows_a, sem_ta, sem_da)
    _wait_add(idx_a, rows_a, sem_ta, sem_da)
    _wait_idx(idx_b, sem_ib)

    plsc.subcore_barrier()

    @pl.when(s == 0)
    def _copy_deg():
        pltpu.sync_copy(deg_sh, deg_out.at[c])

    r0 = s * ROWS_A

    @pl.when(s < NS - 1)
    def _copy_body():
        pltpu.sync_copy(table_sh.at[pl.ds(r0, ROWS_A)],
                        nsum_out.at[c].at[pl.ds(r0, ROWS_A)])

    @pl.when(s == NS - 1)
    def _copy_tail():
        pltpu.sync_copy(table_sh.at[pl.ds(r0, ROWS_LAST)],
                        nsum_out.at[c].at[pl.ds(r0, ROWS_LAST)])


def _dense_body(x_ref, ns_ref, deg_ref, wst_ref, wnt_ref, b_ref, h_ref):
    deg = jnp.maximum(jnp.sum(deg_ref[...], axis=0), 1.0)
    nm = (ns_ref[0] + ns_ref[1]) / deg[:, None]
    h = (jnp.dot(x_ref[...], wst_ref[...], preferred_element_type=_f32)
         + jnp.dot(nm, wnt_ref[...], preferred_element_type=_f32)
         + b_ref[...])
    h_ref[...] = jnp.maximum(h, 0.0)


def _dense(x, nsum, degp, wst, wnt, b2d):
    return pl.pallas_call(
        _dense_body,
        out_shape=jax.ShapeDtypeStruct((N, H), _f32),
    )(x, nsum, degp, wst, wnt, b2d)


@functools.partial(
    pl.kernel,
    out_type=(
        jax.ShapeDtypeStruct((E,), _f32),
        jax.ShapeDtypeStruct((E,), _f32),
    ),
    mesh=plsc.VectorSubcoreMesh(core_axis_name="c", subcore_axis_name="s"),
    scratch_types=[
        pltpu.VMEM((NCH3, C3), _i32),
        pltpu.VMEM((NCH3, C3), _i32),
        pltpu.VMEM((C3, D), _f32),
        pltpu.VMEM((C3, D), _f32),
        pltpu.VMEM((C3, D), _f32),
        pltpu.VMEM((C3, D), _f32),
        pltpu.VMEM((EPW,), _f32),
        pltpu.VMEM((16 * 17,), _f32),
        pltpu.SemaphoreType.DMA,
        pltpu.SemaphoreType.DMA,
        pltpu.SemaphoreType.DMA,
        pltpu.SemaphoreType.DMA,
    ],
    compiler_params=pltpu.CompilerParams(needs_layout_passes=False),
)
def _scores(h_hbm, srcp3_hbm, dstp3_hbm, srcn3_hbm, dstn3_hbm,
            sp_out, sn_out,
            su_v, sv_v, hu_a, hu_b, hv_a, hv_b, out_v, ts_v,
            sem_ua, sem_ub, sem_va, sem_vb):
    c = lax.axis_index("c")
    s = lax.axis_index("s")
    wid = s * NC + c
    base_e = wid * EPW

    def _gather(k, hu_v, hv_v, sem_u, sem_v):
        pltpu.async_copy(h_hbm.at[su_v.at[k]], hu_v, sem_u)
        pltpu.async_copy(h_hbm.at[sv_v.at[k]], hv_v, sem_v)

    def _wait(hu_v, hv_v, sem_u, sem_v):
        pltpu.make_async_copy(h_hbm.at[pl.ds(0, C3)], hu_v, sem_u).wait()
        pltpu.make_async_copy(h_hbm.at[pl.ds(0, C3)], hv_v, sem_v).wait()

    def _compute(k, hu_v, hv_v):
        def _group(g, carry):
            for j in range(16):
                e = g * 16 + j
                acc = hu_v[e, pl.ds(0, 16)] * hv_v[e, pl.ds(0, 16)]
                for q in range(1, D // 16):
                    acc = acc + (hu_v[e, pl.ds(q * 16, 16)]
                                 * hv_v[e, pl.ds(q * 16, 16)])
                ts_v[pl.ds(j * 17, 16)] = acc
            iota17 = lax.broadcasted_iota(_i32, (16,), 0) * 17
            r = plsc.load_gather(ts_v, [iota17])
            for j in range(1, 16):
                r = r + plsc.load_gather(ts_v, [iota17 + j])
            out_v[pl.ds(k * C3 + g * 16, 16)] = r
            return carry
        lax.fori_loop(0, G3, _group, 0)

    for src3_hbm, dst3_hbm, score_out in (
        (srcp3_hbm, dstp3_hbm, sp_out),
        (srcn3_hbm, dstn3_hbm, sn_out),
    ):
        pltpu.sync_copy(src3_hbm.at[wid], su_v)
        pltpu.sync_copy(dst3_hbm.at[wid], sv_v)

        _gather(0, hu_a, hv_a, sem_ua, sem_va)

        def _body(i, carry):
            ka = 2 * i
            _gather(ka + 1, hu_b, hv_b, sem_ub, sem_vb)
            _wait(hu_a, hv_a, sem_ua, sem_va)
            _compute(ka, hu_a, hv_a)
            _gather(ka + 2, hu_a, hv_a, sem_ua, sem_va)
            _wait(hu_b, hv_b, sem_ub, sem_vb)
            _compute(ka + 1, hu_b, hv_b)
            return carry
        lax.fori_loop(0, (NCH3 - 1) // 2, _body, 0)

        _wait(hu_a, hv_a, sem_ua, sem_va)
        _compute(NCH3 - 1, hu_a, hv_a)

        pltpu.sync_copy(out_v, score_out.at[pl.ds(base_e, EPW)])


def kernel(x, edge_index, neg_edge_index, W_self, W_neigh, b):
    src = edge_index[0]
    dst = edge_index[1]
    srcp3 = src.reshape(NW, NCH3, C3)
    dstp3 = dst.reshape(NW, NCH3, C3)
    srcn3 = neg_edge_index[0].reshape(NW, NCH3, C3)
    dstn3 = neg_edge_index[1].reshape(NW, NCH3, C3)
    pad = NCH1P * C1 - EPW
    srcw = jnp.pad(src.reshape(NW, EPW), ((0, 0), (0, pad)),
                   constant_values=N).reshape(NW, NCH1P, C1)
    dstw = jnp.pad(dst.reshape(NW, EPW), ((0, 0), (0, pad)),
                   constant_values=N).reshape(NW, NCH1P, C1)
    rec = jnp.stack([srcw, dstw], axis=2)
    x_ext = jnp.concatenate([x, jnp.zeros((NX - N, D), _f32)], axis=0)
    zeros = jnp.zeros((NX, D), _f32)
    zeros1 = jnp.zeros((NX,), _f32)
    ones = jnp.ones((C1,), _f32)

    nsum, degp = _aggregate(x_ext, rec, zeros, zeros1, ones)
    degp = degp[:, :N]
    h = _dense(x, nsum, degp, W_self.T, W_neigh.T, b.reshape(1, H))
    sp, sn = _scores(h, srcp3, dstp3, srcn3, dstn3)
    return (sp[:, None], sn[:, None])


# speedup vs baseline: 1.2146x; 1.2146x over previous
import functools

import jax
import jax.numpy as jnp
from jax import lax
from jax.experimental import pallas as pl
from jax.experimental.pallas import tpu as pltpu
from jax.experimental.pallas import tpu_sc as plsc

N = 10000
E = 320000
D = 128
H = 128

NC = 2
NS = 16
NW = NC * NS
EPW = E // NW

C1 = 200
NCH1 = EPW // C1
NCH1P = NCH1 + 2
C3 = 80
NCH3 = EPW // C3
G3 = C3 // 16

ROWS_A = 632
ROWS_LAST = N - (NS - 1) * ROWS_A

_f32 = jnp.float32
_i32 = jnp.int32


@functools.partial(
    pl.kernel,
    out_type=(
        jax.ShapeDtypeStruct((NC, N, D), _f32),
        jax.ShapeDtypeStruct((NC, N), _f32),
    ),
    mesh=plsc.VectorSubcoreMesh(core_axis_name="c", subcore_axis_name="s"),
    scratch_types=[
        pltpu.VMEM((C1,), _i32),
        pltpu.VMEM((C1,), _i32),
        pltpu.VMEM((C1,), _i32),
        pltpu.VMEM((C1,), _i32),
        pltpu.VMEM((C1, D), _f32),
        pltpu.VMEM((C1,), _f32),
        pltpu.VMEM_SHARED((N, D), _f32),
        pltpu.VMEM_SHARED((N,), _f32),
        pltpu.SemaphoreType.DMA,
        pltpu.SemaphoreType.DMA,
        pltpu.SemaphoreType.DMA,
        pltpu.SemaphoreType.DMA,
        pltpu.SemaphoreType.DMA,
    ],
    compiler_params=pltpu.CompilerParams(needs_layout_passes=False),
)
def _aggregate(x_hbm, src3_hbm, dst3_hbm, zeros_hbm, zeros1_hbm, ones_hbm,
               nsum_out, deg_out,
               src_a, src_b, dst_a, dst_b, rows_v, ones_v, table_sh, deg_sh,
               sem_sa, sem_sb, sem_da, sem_db, sem_g):
    c = lax.axis_index("c")
    s = lax.axis_index("s")
    wid = s * NC + c

    pltpu.sync_copy(ones_hbm, ones_v)

    @pl.when(s == 0)
    def _init_table():
        pltpu.sync_copy(zeros_hbm, table_sh)

    @pl.when(s == 1)
    def _init_deg():
        pltpu.sync_copy(zeros1_hbm, deg_sh)

    plsc.subcore_barrier()

    def _fetch_idx(k, sidx_v, didx_v, sem_s, sem_d):
        pltpu.async_copy(src3_hbm.at[wid].at[k], sidx_v, sem_s)
        pltpu.async_copy(dst3_hbm.at[wid].at[k], didx_v, sem_d)

    def _wait_idx(sidx_v, didx_v, sem_s, sem_d):
        pltpu.make_async_copy(src3_hbm.at[0].at[0], sidx_v, sem_s).wait()
        pltpu.make_async_copy(dst3_hbm.at[0].at[0], didx_v, sem_d).wait()

    def _process(k, sidx_v, didx_v):
        pltpu.async_copy(x_hbm.at[sidx_v], rows_v, sem_g).wait()
        pltpu.sync_copy(rows_v, table_sh.at[didx_v], add=True)
        pltpu.sync_copy(ones_v, deg_sh.at[didx_v], add=True)

    _fetch_idx(0, src_a, dst_a, sem_sa, sem_da)
    _fetch_idx(1, src_b, dst_b, sem_sb, sem_db)

    def _body(i, carry):
        ka = 2 * i
        _wait_idx(src_a, dst_a, sem_sa, sem_da)
        _process(ka, src_a, dst_a)
        _fetch_idx(ka + 2, src_a, dst_a, sem_sa, sem_da)
        _wait_idx(src_b, dst_b, sem_sb, sem_db)
        _process(ka + 1, src_b, dst_b)
        _fetch_idx(ka + 3, src_b, dst_b, sem_sb, sem_db)
        return carry
    lax.fori_loop(0, NCH1 // 2, _body, 0)

    _wait_idx(src_a, dst_a, sem_sa, sem_da)
    _wait_idx(src_b, dst_b, sem_sb, sem_db)

    plsc.subcore_barrier()

    @pl.when(s == 0)
    def _copy_deg():
        pltpu.sync_copy(deg_sh, deg_out.at[c])

    r0 = s * ROWS_A

    @pl.when(s < NS - 1)
    def _copy_body():
        pltpu.sync_copy(table_sh.at[pl.ds(r0, ROWS_A)],
                        nsum_out.at[c].at[pl.ds(r0, ROWS_A)])

    @pl.when(s == NS - 1)
    def _copy_tail():
        pltpu.sync_copy(table_sh.at[pl.ds(r0, ROWS_LAST)],
                        nsum_out.at[c].at[pl.ds(r0, ROWS_LAST)])


def _dense_body(x_ref, ns_ref, deg_ref, wst_ref, wnt_ref, b_ref, h_ref):
    deg = jnp.maximum(jnp.sum(deg_ref[...], axis=0), 1.0)
    nm = (ns_ref[0] + ns_ref[1]) / deg[:, None]
    h = (jnp.dot(x_ref[...], wst_ref[...], preferred_element_type=_f32)
         + jnp.dot(nm, wnt_ref[...], preferred_element_type=_f32)
         + b_ref[...])
    h_ref[...] = jnp.maximum(h, 0.0)


def _dense(x, nsum, degp, wst, wnt, b2d):
    return pl.pallas_call(
        _dense_body,
        out_shape=jax.ShapeDtypeStruct((N, H), _f32),
    )(x, nsum, degp, wst, wnt, b2d)


@functools.partial(
    pl.kernel,
    out_type=(
        jax.ShapeDtypeStruct((E,), _f32),
        jax.ShapeDtypeStruct((E,), _f32),
    ),
    mesh=plsc.VectorSubcoreMesh(core_axis_name="c", subcore_axis_name="s"),
    scratch_types=[
        pltpu.VMEM((NCH3, C3), _i32),
        pltpu.VMEM((NCH3, C3), _i32),
        pltpu.VMEM((C3, D), _f32),
        pltpu.VMEM((C3, D), _f32),
        pltpu.VMEM((C3, D), _f32),
        pltpu.VMEM((C3, D), _f32),
        pltpu.VMEM((EPW,), _f32),
        pltpu.VMEM((16 * 17,), _f32),
        pltpu.SemaphoreType.DMA,
        pltpu.SemaphoreType.DMA,
        pltpu.SemaphoreType.DMA,
        pltpu.SemaphoreType.DMA,
    ],
    compiler_params=pltpu.CompilerParams(needs_layout_passes=False),
)
def _scores(h_hbm, srcp3_hbm, dstp3_hbm, srcn3_hbm, dstn3_hbm,
            sp_out, sn_out,
            su_v, sv_v, hu_a, hu_b, hv_a, hv_b, out_v, ts_v,
            sem_ua, sem_ub, sem_va, sem_vb):
    c = lax.axis_index("c")
    s = lax.axis_index("s")
    wid = s * NC + c
    base_e = wid * EPW

    def _gather(k, hu_v, hv_v, sem_u, sem_v):
        pltpu.async_copy(h_hbm.at[su_v.at[k]], hu_v, sem_u)
        pltpu.async_copy(h_hbm.at[sv_v.at[k]], hv_v, sem_v)

    def _wait(hu_v, hv_v, sem_u, sem_v):
        pltpu.make_async_copy(h_hbm.at[pl.ds(0, C3)], hu_v, sem_u).wait()
        pltpu.make_async_copy(h_hbm.at[pl.ds(0, C3)], hv_v, sem_v).wait()

    def _compute(k, hu_v, hv_v):
        def _group(g, carry):
            for j in range(16):
                e = g * 16 + j
                acc = hu_v[e, pl.ds(0, 16)] * hv_v[e, pl.ds(0, 16)]
                for q in range(1, D // 16):
                    acc = acc + (hu_v[e, pl.ds(q * 16, 16)]
                                 * hv_v[e, pl.ds(q * 16, 16)])
                ts_v[pl.ds(j * 17, 16)] = acc
            iota17 = lax.broadcasted_iota(_i32, (16,), 0) * 17
            r = plsc.load_gather(ts_v, [iota17])
            for j in range(1, 16):
                r = r + plsc.load_gather(ts_v, [iota17 + j])
            out_v[pl.ds(k * C3 + g * 16, 16)] = r
            return carry
        lax.fori_loop(0, G3, _group, 0)

    for src3_hbm, dst3_hbm, score_out in (
        (srcp3_hbm, dstp3_hbm, sp_out),
        (srcn3_hbm, dstn3_hbm, sn_out),
    ):
        pltpu.sync_copy(src3_hbm.at[wid], su_v)
        pltpu.sync_copy(dst3_hbm.at[wid], sv_v)

        _gather(0, hu_a, hv_a, sem_ua, sem_va)

        def _body(i, carry):
            ka = 2 * i
            _gather(ka + 1, hu_b, hv_b, sem_ub, sem_vb)
            _wait(hu_a, hv_a, sem_ua, sem_va)
            _compute(ka, hu_a, hv_a)
            _gather(ka + 2, hu_a, hv_a, sem_ua, sem_va)
            _wait(hu_b, hv_b, sem_ub, sem_vb)
            _compute(ka + 1, hu_b, hv_b)
            return carry
        lax.fori_loop(0, (NCH3 - 1) // 2, _body, 0)

        _wait(hu_a, hv_a, sem_ua, sem_va)
        _compute(NCH3 - 1, hu_a, hv_a)

        pltpu.sync_copy(out_v, score_out.at[pl.ds(base_e, EPW)])


def kernel(x, edge_index, neg_edge_index, W_self, W_neigh, b):
    src = edge_index[0]
    dst = edge_index[1]
    srcp3 = src.reshape(NW, NCH3, C3)
    dstp3 = dst.reshape(NW, NCH3, C3)
    srcn3 = neg_edge_index[0].reshape(NW, NCH3, C3)
    dstn3 = neg_edge_index[1].reshape(NW, NCH3, C3)
    pad = NCH1P * C1 - EPW
    src3 = jnp.pad(src.reshape(NW, EPW), ((0, 0), (0, pad))
                   ).reshape(NW, NCH1P, C1)
    dst3 = jnp.pad(dst.reshape(NW, EPW), ((0, 0), (0, pad))
                   ).reshape(NW, NCH1P, C1)
    zeros = jnp.zeros((N, D), _f32)
    zeros1 = jnp.zeros((N,), _f32)
    ones = jnp.ones((C1,), _f32)

    nsum, degp = _aggregate(x, src3, dst3, zeros, zeros1, ones)
    h = _dense(x, nsum, degp, W_self.T, W_neigh.T, b.reshape(1, H))
    sp, sn = _scores(h, srcp3, dstp3, srcn3, dstn3)
    return (sp[:, None], sn[:, None])
